# baseline (speedup 1.0000x reference)
import functools
import math

import jax
import jax.numpy as jnp
from jax import lax
from jax.experimental import pallas as pl
from jax.experimental.pallas import tpu as pltpu
from jax.experimental.pallas import tpu_sc as plsc


def _make_gather_scale(V, D, B, scale):
    info = plsc.get_sparse_core_info()
    NC, NS, L = info.num_cores, info.num_subcores, info.num_lanes
    NW = NC * NS
    assert B % NW == 0 and D % L == 0
    BPW = B // NW
    K = 16
    assert BPW % K == 0
    NCH = BPW // K

    mesh = plsc.VectorSubcoreMesh(core_axis_name="c", subcore_axis_name="s")

    @functools.partial(
        pl.kernel,
        mesh=mesh,
        out_type=jax.ShapeDtypeStruct((B, D), jnp.float32),
        scratch_types=[
            pltpu.VMEM((BPW,), jnp.int32),
            pltpu.VMEM((K, D), jnp.float32),
            pltpu.SemaphoreType.DMA,
        ],
    )
    def k(table_hbm, idx_hbm, out_hbm, idx_v, rows_v, sem):
        wid = lax.axis_index("s") * NC + lax.axis_index("c")
        base = wid * BPW
        pltpu.sync_copy(idx_hbm.at[pl.ds(base, BPW)], idx_v)

        def chunk_body(c, carry):
            pltpu.async_copy(
                table_hbm.at[idx_v.at[pl.ds(c * K, K)]], rows_v, sem
            ).wait()

            def scale_body(i, carry2):
                r = i // (D // L)
                j = (i % (D // L)) * L
                rows_v[r, pl.ds(j, L)] = rows_v[r, pl.ds(j, L)] * scale
                return carry2

            lax.fori_loop(0, K * (D // L), scale_body, 0, unroll=4)

            pltpu.sync_copy(rows_v, out_hbm.at[pl.ds(base + c * K, K)])
            return carry

        lax.fori_loop(0, NCH, chunk_body, 0)

    return k


def kernel(sequence, table):
    Bt, S = sequence.shape
    V, D = table.shape
    B = Bt * S
    idx = sequence.reshape(B).astype(jnp.int32)
    scale = jnp.float32(math.sqrt(D))
    out = _make_gather_scale(V, D, B, scale)(table, idx)
    return out.reshape(Bt, S, D)


# speedup vs baseline: 1.8128x; 1.8128x over previous
import functools
import math

import jax
import jax.numpy as jnp
from jax import lax
from jax.experimental import pallas as pl
from jax.experimental.pallas import tpu as pltpu
from jax.experimental.pallas import tpu_sc as plsc


def _make_gather_scale(V, D, B, scale):
    info = plsc.get_sparse_core_info()
    NC, NS, L = info.num_cores, info.num_subcores, info.num_lanes
    NW = NC * NS
    assert B % NW == 0 and D % L == 0
    BPW = B // NW
    K = 16
    assert BPW % (2 * K) == 0
    NCH = BPW // K
    NP = NCH // 2

    mesh = plsc.VectorSubcoreMesh(core_axis_name="c", subcore_axis_name="s")

    @functools.partial(
        pl.kernel,
        mesh=mesh,
        out_type=jax.ShapeDtypeStruct((B, D), jnp.float32),
        scratch_types=[
            pltpu.VMEM((BPW,), jnp.int32),
            pltpu.VMEM((K, D), jnp.float32),
            pltpu.VMEM((K, D), jnp.float32),
            pltpu.SemaphoreType.DMA,
            pltpu.SemaphoreType.DMA,
            pltpu.SemaphoreType.DMA,
            pltpu.SemaphoreType.DMA,
        ],
    )
    def k(table_hbm, idx_hbm, out_hbm, idx_v, r0, r1, sg0, sg1, ss0, ss1):
        wid = lax.axis_index("s") * NC + lax.axis_index("c")
        base = wid * BPW
        pltpu.sync_copy(idx_hbm.at[pl.ds(base, BPW)], idx_v)

        def start_g(c, buf, sem):
            pltpu.async_copy(table_hbm.at[idx_v.at[pl.ds(c * K, K)]], buf, sem)

        def wait_g(buf, sem):
            pltpu.make_async_copy(table_hbm.at[pl.ds(0, K)], buf, sem).wait()

        def start_s(c, buf, sem):
            pltpu.async_copy(buf, out_hbm.at[pl.ds(base + c * K, K)], sem)

        def wait_s(buf, sem):
            pltpu.make_async_copy(buf, out_hbm.at[pl.ds(base, K)], sem).wait()

        def scale_buf(buf):
            def row(r, carry):
                def col(j, carry2):
                    buf[r, pl.ds(j * L, L)] = buf[r, pl.ds(j * L, L)] * scale
                    return carry2
                lax.fori_loop(0, D // L, col, 0, unroll=8)
                return carry
            lax.fori_loop(0, K, row, 0)

        start_g(0, r0, sg0)

        def pair(p, carry):
            c0 = 2 * p
            wait_g(r0, sg0)

            @pl.when(p > 0)
            def _():
                wait_s(r1, ss1)
            start_g(c0 + 1, r1, sg1)
            scale_buf(r0)
            start_s(c0, r0, ss0)

            wait_g(r1, sg1)

            @pl.when(p < NP - 1)
            def _():
                wait_s(r0, ss0)
                start_g(c0 + 2, r0, sg0)
            scale_buf(r1)
            start_s(c0 + 1, r1, ss1)
            return carry

        lax.fori_loop(0, NP, pair, 0)
        wait_s(r0, ss0)
        wait_s(r1, ss1)

    return k


def kernel(sequence, table):
    Bt, S = sequence.shape
    V, D = table.shape
    B = Bt * S
    idx = sequence.reshape(B).astype(jnp.int32)
    scale = jnp.float32(math.sqrt(D))
    out = _make_gather_scale(V, D, B, scale)(table, idx)
    return out.reshape(Bt, S, D)


# speedup vs baseline: 1.8173x; 1.0025x over previous
import functools
import math

import jax
import jax.numpy as jnp
from jax import lax
from jax.experimental import pallas as pl
from jax.experimental.pallas import tpu as pltpu
from jax.experimental.pallas import tpu_sc as plsc

_NBUF = 4


def _make_gather_scale(V, D, B, scale):
    info = plsc.get_sparse_core_info()
    NC, NS, L = info.num_cores, info.num_subcores, info.num_lanes
    NW = NC * NS
    assert B % NW == 0 and D % L == 0
    BPW = B // NW
    K = 8
    assert BPW % (_NBUF * K) == 0
    NCH = BPW // K
    NG = NCH // _NBUF

    mesh = plsc.VectorSubcoreMesh(core_axis_name="c", subcore_axis_name="s")

    @functools.partial(
        pl.kernel,
        mesh=mesh,
        out_type=jax.ShapeDtypeStruct((B, D), jnp.float32),
        scratch_types=[
            pltpu.VMEM((BPW,), jnp.int32),
            *[pltpu.VMEM((K, D), jnp.float32) for _ in range(_NBUF)],
            *[pltpu.SemaphoreType.DMA for _ in range(2 * _NBUF)],
        ],
    )
    def k(table_hbm, idx_hbm, out_hbm, idx_v, *bufs_and_sems):
        bufs = bufs_and_sems[:_NBUF]
        sg = bufs_and_sems[_NBUF:2 * _NBUF]
        ss = bufs_and_sems[2 * _NBUF:]
        wid = lax.axis_index("s") * NC + lax.axis_index("c")
        base = wid * BPW
        pltpu.sync_copy(idx_hbm.at[pl.ds(base, BPW)], idx_v)

        def start_g(c, j):
            pltpu.async_copy(
                table_hbm.at[idx_v.at[pl.ds(c * K, K)]], bufs[j], sg[j]
            )

        def wait_g(j):
            pltpu.make_async_copy(table_hbm.at[pl.ds(0, K)], bufs[j], sg[j]).wait()

        def start_s(c, j):
            pltpu.async_copy(bufs[j], out_hbm.at[pl.ds(base + c * K, K)], ss[j])

        def wait_s(j):
            pltpu.make_async_copy(bufs[j], out_hbm.at[pl.ds(base, K)], ss[j]).wait()

        def scale_buf(buf):
            def row(r, carry):
                def col(jj, carry2):
                    buf[r, pl.ds(jj * L, L)] = buf[r, pl.ds(jj * L, L)] * scale
                    return carry2
                lax.fori_loop(0, D // L, col, 0, unroll=8)
                return carry
            lax.fori_loop(0, K, row, 0)

        for j in range(_NBUF - 1):
            start_g(j, j)

        def group(g, carry):
            for j in range(_NBUF):
                c = g * _NBUF + j
                tgt = (j + _NBUF - 1) % _NBUF
                wait_g(j)
                if j == 0:
                    @pl.when(g > 0)
                    def _():
                        wait_s(tgt)
                else:
                    wait_s(tgt)

                @pl.when(c + _NBUF - 1 < NCH)
                def _():
                    start_g(c + _NBUF - 1, tgt)
                scale_buf(bufs[j])
                start_s(c, j)
            return carry

        lax.fori_loop(0, NG, group, 0)
        wait_s(_NBUF - 1)

    return k


def kernel(sequence, table):
    Bt, S = sequence.shape
    V, D = table.shape
    B = Bt * S
    idx = sequence.reshape(B).astype(jnp.int32)
    scale = jnp.float32(math.sqrt(D))
    out = _make_gather_scale(V, D, B, scale)(table, idx)
    return out.reshape(Bt, S, D)
